# baseline (speedup 1.0000x reference)
import functools

import jax
import jax.numpy as jnp
from jax.experimental import pallas as pl
from jax.experimental.pallas import tpu as pltpu

NUM_EMBEDDINGS = 8192
EMBEDDING_DIM = 32
COMMITMENT_COST = 0.25
N_ROWS = 8192
BLOCK_ROWS = 512
N_BLOCKS = N_ROWS // BLOCK_ROWS


def _vq_kernel(z_ref, emb_ref, idx_ref, enc_ref, quant_ref,
               loss_ref, perp_ref, counts_ref, loss_acc_ref):
    i = pl.program_id(0)

    zb = z_ref[...]
    emb = emb_ref[...]
    idx = idx_ref[...]

    code_iota = jax.lax.broadcasted_iota(
        jnp.int32, (BLOCK_ROWS, NUM_EMBEDDINGS), 1)
    onehot = (code_iota == idx).astype(jnp.float32)
    enc_ref[...] = onehot

    quant = jax.lax.dot_general(
        onehot, emb,
        dimension_numbers=(((1,), (0,)), ((), ())),
        precision=jax.lax.Precision.HIGHEST,
        preferred_element_type=jnp.float32)
    quant_ref[...] = quant

    diff = quant - zb
    block_loss = jnp.sum(diff * diff)
    block_counts = jnp.sum(onehot, axis=0, keepdims=True)

    @pl.when(i == 0)
    def _init():
        loss_acc_ref[0, 0] = block_loss
        counts_ref[...] = block_counts

    @pl.when(i > 0)
    def _acc():
        loss_acc_ref[0, 0] += block_loss
        counts_ref[...] += block_counts

    @pl.when(i == N_BLOCKS - 1)
    def _finish():
        loss_ref[...] = jnp.full(
            (1, 1),
            (COMMITMENT_COST / (N_ROWS * EMBEDDING_DIM)) * loss_acc_ref[0, 0],
            dtype=jnp.float32)
        avg_probs = counts_ref[...] * (1.0 / N_ROWS)
        perp_ref[...] = jnp.exp(
            -jnp.sum(avg_probs * jnp.log(avg_probs + 1e-10),
                     keepdims=True)).reshape(1, 1)


@jax.jit
def kernel(z_e, embedding):
    B, C, H, W = z_e.shape
    z = jnp.transpose(z_e, (0, 2, 3, 1)).reshape(-1, EMBEDDING_DIM)

    z2 = jnp.sum(z ** 2, axis=1, keepdims=True)
    e2 = jnp.sum(embedding ** 2, axis=1)
    dist = (z2 + e2) - 2.0 * (z @ embedding.T)
    idx = jnp.argmin(dist, axis=1).astype(jnp.int32)[:, None]

    enc, quant, loss, perp = pl.pallas_call(
        _vq_kernel,
        grid=(N_BLOCKS,),
        in_specs=[
            pl.BlockSpec((BLOCK_ROWS, EMBEDDING_DIM), lambda i: (i, 0)),
            pl.BlockSpec((NUM_EMBEDDINGS, EMBEDDING_DIM), lambda i: (0, 0)),
            pl.BlockSpec((BLOCK_ROWS, 1), lambda i: (i, 0)),
        ],
        out_specs=[
            pl.BlockSpec((BLOCK_ROWS, NUM_EMBEDDINGS), lambda i: (i, 0)),
            pl.BlockSpec((BLOCK_ROWS, EMBEDDING_DIM), lambda i: (i, 0)),
            pl.BlockSpec((1, 1), lambda i: (0, 0)),
            pl.BlockSpec((1, 1), lambda i: (0, 0)),
        ],
        out_shape=[
            jax.ShapeDtypeStruct((N_ROWS, NUM_EMBEDDINGS), jnp.float32),
            jax.ShapeDtypeStruct((N_ROWS, EMBEDDING_DIM), jnp.float32),
            jax.ShapeDtypeStruct((1, 1), jnp.float32),
            jax.ShapeDtypeStruct((1, 1), jnp.float32),
        ],
        scratch_shapes=[
            pltpu.VMEM((1, NUM_EMBEDDINGS), jnp.float32),
            pltpu.SMEM((1, 1), jnp.float32),
        ],
    )(z, embedding, idx)

    quantized_out = jnp.transpose(
        quant.reshape(B, H, W, C), (0, 3, 1, 2))
    return quantized_out, loss[0, 0], perp[0, 0], enc


# speedup vs baseline: 1.6074x; 1.6074x over previous
import functools

import jax
import jax.numpy as jnp
from jax.experimental import pallas as pl
from jax.experimental.pallas import tpu as pltpu

NUM_EMBEDDINGS = 8192
EMBEDDING_DIM = 32
COMMITMENT_COST = 0.25
N_ROWS = 8192
BLOCK_ROWS = 512
N_BLOCKS = N_ROWS // BLOCK_ROWS


def _vq_kernel(z_ref, emb_ref, idx_ref, enc_ref, quant_ref,
               loss_ref, perp_ref, counts_ref, loss_acc_ref):
    i = pl.program_id(0)

    zb = z_ref[...]
    emb = emb_ref[...]
    idx = idx_ref[...]

    code_iota = jax.lax.broadcasted_iota(
        jnp.int32, (BLOCK_ROWS, NUM_EMBEDDINGS), 1)
    onehot = (code_iota == idx).astype(jnp.float32)
    enc_ref[...] = onehot

    quant = jax.lax.dot_general(
        onehot, emb,
        dimension_numbers=(((1,), (0,)), ((), ())),
        preferred_element_type=jnp.float32)
    quant_ref[...] = quant

    diff = quant - zb
    block_loss = jnp.sum(diff * diff)
    ones_row = jnp.ones((1, BLOCK_ROWS), jnp.float32)
    block_counts = jax.lax.dot_general(
        ones_row, onehot,
        dimension_numbers=(((1,), (0,)), ((), ())),
        preferred_element_type=jnp.float32)

    @pl.when(i == 0)
    def _init():
        loss_acc_ref[0, 0] = block_loss
        counts_ref[...] = block_counts

    @pl.when(i > 0)
    def _acc():
        loss_acc_ref[0, 0] += block_loss
        counts_ref[...] += block_counts

    @pl.when(i == N_BLOCKS - 1)
    def _finish():
        loss_ref[...] = jnp.full(
            (1, 1),
            (COMMITMENT_COST / (N_ROWS * EMBEDDING_DIM)) * loss_acc_ref[0, 0],
            dtype=jnp.float32)
        avg_probs = counts_ref[...] * (1.0 / N_ROWS)
        perp_ref[...] = jnp.exp(
            -jnp.sum(avg_probs * jnp.log(avg_probs + 1e-10),
                     keepdims=True)).reshape(1, 1)


@jax.jit
def kernel(z_e, embedding):
    B, C, H, W = z_e.shape
    z = jnp.transpose(z_e, (0, 2, 3, 1)).reshape(-1, EMBEDDING_DIM)

    z2 = jnp.sum(z ** 2, axis=1, keepdims=True)
    e2 = jnp.sum(embedding ** 2, axis=1)
    dist = (z2 + e2) - 2.0 * (z @ embedding.T)
    idx = jnp.argmin(dist, axis=1).astype(jnp.int32)[:, None]

    enc, quant, loss, perp = pl.pallas_call(
        _vq_kernel,
        grid=(N_BLOCKS,),
        in_specs=[
            pl.BlockSpec((BLOCK_ROWS, EMBEDDING_DIM), lambda i: (i, 0)),
            pl.BlockSpec((NUM_EMBEDDINGS, EMBEDDING_DIM), lambda i: (0, 0)),
            pl.BlockSpec((BLOCK_ROWS, 1), lambda i: (i, 0)),
        ],
        out_specs=[
            pl.BlockSpec((BLOCK_ROWS, NUM_EMBEDDINGS), lambda i: (i, 0)),
            pl.BlockSpec((BLOCK_ROWS, EMBEDDING_DIM), lambda i: (i, 0)),
            pl.BlockSpec((1, 1), lambda i: (0, 0)),
            pl.BlockSpec((1, 1), lambda i: (0, 0)),
        ],
        out_shape=[
            jax.ShapeDtypeStruct((N_ROWS, NUM_EMBEDDINGS), jnp.float32),
            jax.ShapeDtypeStruct((N_ROWS, EMBEDDING_DIM), jnp.float32),
            jax.ShapeDtypeStruct((1, 1), jnp.float32),
            jax.ShapeDtypeStruct((1, 1), jnp.float32),
        ],
        scratch_shapes=[
            pltpu.VMEM((1, NUM_EMBEDDINGS), jnp.float32),
            pltpu.SMEM((1, 1), jnp.float32),
        ],
    )(z, embedding, idx)

    quantized_out = jnp.transpose(
        quant.reshape(B, H, W, C), (0, 3, 1, 2))
    return quantized_out, loss[0, 0], perp[0, 0], enc
